# baseline (speedup 1.0000x reference)
import functools

import numpy as np
import jax
import jax.numpy as jnp
from jax import lax
from jax.experimental import pallas as pl
from jax.experimental.pallas import tpu as pltpu
from jax.experimental.pallas import tpu_sc as plsc

NUM_CORES = 2
NUM_SUBCORES = 16
NW = NUM_CORES * NUM_SUBCORES

BATCH = 16384
HIST = 50
HIST_PAD = 56
PAIR = 4
IDX_PER_DMA = HIST_PAD * PAIR
DIM = 32
RING = 8

ROWS_PER_W = BATCH // NW
PAIRS_PER_W = ROWS_PER_W // PAIR

_RSQRT_TAB = np.zeros((64,), np.float32)
_RSQRT_TAB[: HIST + 1] = (
    1.0 / np.sqrt(np.maximum(np.arange(HIST + 1, dtype=np.float64), 1.0))
).astype(np.float32)


def _emb_bag_body(idx_hbm, table_hbm, rtab_hbm, out_hbm,
                  idx_v, gbuf, out_v, rtab_v, s0, s1, s2, s3, s4, s5, s6, s7):
    sems = (s0, s1, s2, s3, s4, s5, s6, s7)
    wid = lax.axis_index("s") * NUM_CORES + lax.axis_index("c")
    pair_base = wid * PAIRS_PER_W
    row_base = wid * ROWS_PER_W

    pltpu.sync_copy(idx_hbm.at[pl.ds(pair_base, PAIRS_PER_W)], idx_v)
    pltpu.sync_copy(rtab_hbm, rtab_v)

    def start(p, b):
        pltpu.make_async_copy(
            table_hbm.at[idx_v.at[p]], gbuf.at[b], sems[b]
        ).start()

    def wait(b):
        pltpu.make_async_copy(
            table_hbm.at[idx_v.at[0]], gbuf.at[b], sems[b]
        ).wait()

    lane = lax.iota(jnp.int32, 16)
    tail = lane >= 8

    def compute(p, b):
        for r in range(PAIR):
            off = r * HIST_PAD
            i0 = idx_v[p, pl.ds(off, 16)]
            i1 = idx_v[p, pl.ds(off + 16, 16)]
            i2 = idx_v[p, pl.ds(off + 32, 16)]
            i3 = idx_v[p, pl.ds(off + 40, 16)]
            tail_m = lax.iota(jnp.int32, 16) >= 8
            c_vec = (
                plsc.all_reduce_population_count(i0 != 0)
                + plsc.all_reduce_population_count(i1 != 0)
                + plsc.all_reduce_population_count(i2 != 0)
                + plsc.all_reduce_population_count((i3 != 0) & tail_m)
            )
            scale = plsc.load_gather(rtab_v, [c_vec])

            acc = [jnp.zeros((16,), jnp.float32) for _ in range(4)]
            for l in range(HIST_PAD):
                row = off + l
                j = l & 1
                acc[j] = acc[j] + gbuf[b, row, pl.ds(0, 16)]
                acc[2 + j] = acc[2 + j] + gbuf[b, row, pl.ds(16, 16)]

            out_row = p * PAIR + r
            out_v[out_row, pl.ds(0, 16)] = (acc[0] + acc[1]) * scale
            out_v[out_row, pl.ds(16, 16)] = (acc[2] + acc[3]) * scale

    for b in range(RING):
        start(b, b)

    def loop_body(i, carry):
        p0 = i * RING
        for b in range(RING):
            p = p0 + b
            wait(b)
            compute(p, b)

            @pl.when(p + RING < PAIRS_PER_W)
            def _():
                start(p + RING, b)

        return carry

    lax.fori_loop(0, PAIRS_PER_W // RING, loop_body, 0)

    pltpu.sync_copy(out_v, out_hbm.at[pl.ds(row_base, ROWS_PER_W)])


_emb_bag = functools.partial(
    pl.kernel,
    out_type=jax.ShapeDtypeStruct((BATCH, DIM), jnp.float32),
    mesh=plsc.VectorSubcoreMesh(core_axis_name="c", subcore_axis_name="s"),
    compiler_params=pltpu.CompilerParams(
        use_tc_tiling_on_sc=False, needs_layout_passes=False
    ),
    scratch_types=[
        pltpu.VMEM((PAIRS_PER_W, IDX_PER_DMA), jnp.int32),
        pltpu.VMEM((RING, IDX_PER_DMA, DIM), jnp.float32),
        pltpu.VMEM((ROWS_PER_W, DIM), jnp.float32),
        pltpu.VMEM((64,), jnp.float32),
        pltpu.SemaphoreType.DMA,
        pltpu.SemaphoreType.DMA,
        pltpu.SemaphoreType.DMA,
        pltpu.SemaphoreType.DMA,
        pltpu.SemaphoreType.DMA,
        pltpu.SemaphoreType.DMA,
        pltpu.SemaphoreType.DMA,
        pltpu.SemaphoreType.DMA,
    ],
)(_emb_bag_body)


def kernel(input, W):
    idx = jnp.pad(input.astype(jnp.int32), ((0, 0), (0, HIST_PAD - HIST)))
    idx_pairs = idx.reshape(BATCH // PAIR, IDX_PER_DMA)
    rtab = jnp.asarray(_RSQRT_TAB)
    return _emb_bag(idx_pairs, W, rtab)


# speedup vs baseline: 2.4622x; 2.4622x over previous
import functools

import numpy as np
import jax
import jax.numpy as jnp
from jax import lax
from jax.experimental import pallas as pl
from jax.experimental.pallas import tpu as pltpu
from jax.experimental.pallas import tpu_sc as plsc

NUM_CORES = 2
NUM_SUBCORES = 16
NW = NUM_CORES * NUM_SUBCORES

BATCH = 16384
HIST = 50
HIST_PAD = 56
GROUP = 4
IDX_PER_DMA = HIST * GROUP
DIM = 32
RING = 4

ROWS_PER_W = BATCH // NW
GROUPS_PER_W = ROWS_PER_W // GROUP

_RSQRT_TAB = np.zeros((64,), np.float32)
_RSQRT_TAB[: HIST + 1] = (
    1.0 / np.sqrt(np.maximum(np.arange(HIST + 1, dtype=np.float64), 1.0))
).astype(np.float32)


def _emb_bag_body(idx_g_hbm, idx_c_hbm, table_hbm, rtab_hbm, out_hbm,
                  idx_v, idx_c_v, gbuf, out_v, rtab_v, s0, s1, s2, s3):
    sems = (s0, s1, s2, s3)
    wid = lax.axis_index("s") * NUM_CORES + lax.axis_index("c")
    group_base = wid * GROUPS_PER_W
    row_base = wid * ROWS_PER_W

    pltpu.sync_copy(idx_g_hbm.at[pl.ds(group_base, GROUPS_PER_W)], idx_v)
    pltpu.sync_copy(idx_c_hbm.at[pl.ds(row_base, ROWS_PER_W)], idx_c_v)
    pltpu.sync_copy(rtab_hbm, rtab_v)

    def start(p, b):
        pltpu.make_async_copy(
            table_hbm.at[idx_v.at[p]], gbuf.at[b], sems[b]
        ).start()

    def wait(b):
        pltpu.make_async_copy(
            table_hbm.at[idx_v.at[0]], gbuf.at[b], sems[b]
        ).wait()

    def compute(p, b):
        for r in range(GROUP):
            out_row = p * GROUP + r
            i0 = idx_c_v[out_row, pl.ds(0, 16)]
            i1 = idx_c_v[out_row, pl.ds(16, 16)]
            i2 = idx_c_v[out_row, pl.ds(32, 16)]
            i3 = idx_c_v[out_row, pl.ds(40, 16)]
            tail_m = lax.iota(jnp.int32, 16) >= 8
            c_vec = (
                plsc.all_reduce_population_count(i0 != 0)
                + plsc.all_reduce_population_count(i1 != 0)
                + plsc.all_reduce_population_count(i2 != 0)
                + plsc.all_reduce_population_count((i3 != 0) & tail_m)
            )
            scale = plsc.load_gather(rtab_v, [c_vec])

            acc = [jnp.zeros((16,), jnp.float32) for _ in range(4)]
            for l in range(HIST):
                row = r * HIST + l
                j = l & 1
                acc[j] = acc[j] + gbuf[b, row, pl.ds(0, 16)]
                acc[2 + j] = acc[2 + j] + gbuf[b, row, pl.ds(16, 16)]

            out_v[out_row, pl.ds(0, 16)] = (acc[0] + acc[1]) * scale
            out_v[out_row, pl.ds(16, 16)] = (acc[2] + acc[3]) * scale

    for b in range(RING):
        start(b, b)

    def loop_body(i, carry):
        p0 = i * RING
        for b in range(RING):
            p = p0 + b
            wait(b)
            compute(p, b)

            @pl.when(p + RING < GROUPS_PER_W)
            def _():
                start(p + RING, b)

        return carry

    lax.fori_loop(0, GROUPS_PER_W // RING, loop_body, 0)

    pltpu.sync_copy(out_v, out_hbm.at[pl.ds(row_base, ROWS_PER_W)])


_emb_bag = functools.partial(
    pl.kernel,
    out_type=jax.ShapeDtypeStruct((BATCH, DIM), jnp.float32),
    mesh=plsc.VectorSubcoreMesh(core_axis_name="c", subcore_axis_name="s"),
    compiler_params=pltpu.CompilerParams(
        use_tc_tiling_on_sc=False, needs_layout_passes=False
    ),
    scratch_types=[
        pltpu.VMEM((GROUPS_PER_W, IDX_PER_DMA), jnp.int32),
        pltpu.VMEM((ROWS_PER_W, HIST_PAD), jnp.int32),
        pltpu.VMEM((RING, IDX_PER_DMA, DIM), jnp.float32),
        pltpu.VMEM((ROWS_PER_W, DIM), jnp.float32),
        pltpu.VMEM((64,), jnp.float32),
        pltpu.SemaphoreType.DMA,
        pltpu.SemaphoreType.DMA,
        pltpu.SemaphoreType.DMA,
        pltpu.SemaphoreType.DMA,
    ],
)(_emb_bag_body)


def kernel(input, W):
    idx = input.astype(jnp.int32)
    idx_groups = idx.reshape(BATCH // GROUP, IDX_PER_DMA)
    idx_cnt = jnp.pad(idx, ((0, 0), (0, HIST_PAD - HIST)))
    rtab = jnp.asarray(_RSQRT_TAB)
    return _emb_bag(idx_groups, idx_cnt, W, rtab)


# speedup vs baseline: 2.4630x; 1.0003x over previous
import functools

import numpy as np
import jax
import jax.numpy as jnp
from jax import lax
from jax.experimental import pallas as pl
from jax.experimental.pallas import tpu as pltpu
from jax.experimental.pallas import tpu_sc as plsc

NUM_CORES = 2
NUM_SUBCORES = 16
NW = NUM_CORES * NUM_SUBCORES

BATCH = 16384
NUM_EMB = 1000000
HIST = 50
HIST_PAD = 56
GROUP = 4
IDX_PER_DMA = HIST * GROUP
DIM = 32
RING = 4

ROWS_PER_W = BATCH // NW
GROUPS_PER_W = ROWS_PER_W // GROUP

_RSQRT_TAB = np.zeros((64,), np.float32)
_RSQRT_TAB[: HIST + 1] = (
    1.0 / np.sqrt(np.maximum(np.arange(HIST + 1, dtype=np.float64), 1.0))
).astype(np.float32)


def _emb_bag_body(idx_g_hbm, idx_c_hbm, table_hbm, rtab_hbm, out_hbm,
                  idx_v, idx_c_v, gbuf, out_v, rtab_v, s0, s1, s2, s3):
    sems = (s0, s1, s2, s3)
    wid = lax.axis_index("s") * NUM_CORES + lax.axis_index("c")
    group_base = wid * GROUPS_PER_W
    row_base = wid * ROWS_PER_W

    pltpu.sync_copy(idx_g_hbm.at[pl.ds(group_base, GROUPS_PER_W)], idx_v)
    pltpu.sync_copy(idx_c_hbm.at[pl.ds(row_base, ROWS_PER_W)], idx_c_v)
    pltpu.sync_copy(rtab_hbm, rtab_v)

    def start(p, b):
        pltpu.make_async_copy(
            table_hbm.at[idx_v.at[p]], gbuf.at[b], sems[b]
        ).start()

    def wait(b):
        pltpu.make_async_copy(
            table_hbm.at[idx_v.at[0]], gbuf.at[b], sems[b]
        ).wait()

    def compute(p, b):
        for r in range(GROUP):
            out_row = p * GROUP + r
            i0 = idx_c_v[out_row, pl.ds(0, 16)]
            i1 = idx_c_v[out_row, pl.ds(16, 16)]
            i2 = idx_c_v[out_row, pl.ds(32, 16)]
            i3 = idx_c_v[out_row, pl.ds(40, 16)]
            tail_m = lax.iota(jnp.int32, 16) >= 8
            c_vec = (
                plsc.all_reduce_population_count(i0 != 0)
                + plsc.all_reduce_population_count(i1 != 0)
                + plsc.all_reduce_population_count(i2 != 0)
                + plsc.all_reduce_population_count((i3 != 0) & tail_m)
            )
            scale = plsc.load_gather(rtab_v, [c_vec])

            acc = [jnp.zeros((16,), jnp.float32) for _ in range(4)]
            for l in range(HIST):
                row = r * HIST + l
                j = l & 1
                acc[j] = acc[j] + gbuf[b, row, pl.ds(0, 16)]
                acc[2 + j] = acc[2 + j] + gbuf[b, row, pl.ds(16, 16)]

            out_v[out_row, pl.ds(0, 16)] = (acc[0] + acc[1]) * scale
            out_v[out_row, pl.ds(16, 16)] = (acc[2] + acc[3]) * scale

    for b in range(RING):
        start(b, b)

    def loop_body(i, carry):
        p0 = i * RING
        for b in range(RING):
            p = p0 + b
            wait(b)
            compute(p, b)

            @pl.when(p + RING < GROUPS_PER_W)
            def _():
                start(p + RING, b)

        return carry

    lax.fori_loop(0, GROUPS_PER_W // RING, loop_body, 0)

    pltpu.sync_copy(out_v, out_hbm.at[pl.ds(row_base, ROWS_PER_W)])


_emb_bag = functools.partial(
    pl.kernel,
    out_type=jax.ShapeDtypeStruct((BATCH, DIM), jnp.float32),
    mesh=plsc.VectorSubcoreMesh(core_axis_name="c", subcore_axis_name="s"),
    compiler_params=pltpu.CompilerParams(
        use_tc_tiling_on_sc=False, needs_layout_passes=False
    ),
    scratch_types=[
        pltpu.VMEM((GROUPS_PER_W, IDX_PER_DMA), jnp.int32),
        pltpu.VMEM((ROWS_PER_W, HIST_PAD), jnp.int32),
        pltpu.VMEM((RING, IDX_PER_DMA, DIM), jnp.float32),
        pltpu.VMEM((ROWS_PER_W, DIM), jnp.float32),
        pltpu.VMEM((64,), jnp.float32),
        pltpu.SemaphoreType.DMA,
        pltpu.SemaphoreType.DMA,
        pltpu.SemaphoreType.DMA,
        pltpu.SemaphoreType.DMA,
    ],
)(_emb_bag_body)


def kernel(input, W):
    idx = input.astype(jnp.int32)
    idx_groups = idx.reshape(BATCH // GROUP, IDX_PER_DMA)
    idx_cnt = jnp.pad(idx, ((0, 0), (0, HIST_PAD - HIST)))
    rtab = jnp.asarray(_RSQRT_TAB)
    w_flat = jax.lax.optimization_barrier(jnp.reshape(W, (NUM_EMB * DIM,)))
    w_lin = jnp.reshape(w_flat, (NUM_EMB, DIM))
    return _emb_bag(idx_groups, idx_cnt, w_lin, rtab)
